# baseline (speedup 1.0000x reference)
import functools

import jax
import jax.numpy as jnp
from jax import lax
from jax.experimental import pallas as pl
from jax.experimental.pallas import tpu as pltpu
from jax.experimental.pallas import tpu_sc as plsc

PH = 16
DH = 1024
NROWS = PH * PH
D = 2 * DH
NC = 2
NS = 16
RPW = NROWS // (NC * NS)
L = 16

_mesh = plsc.VectorSubcoreMesh(core_axis_name="c", subcore_axis_name="s")


@functools.partial(
    pl.kernel,
    mesh=_mesh,
    out_type=jax.ShapeDtypeStruct((NROWS, D), jnp.float32),
    scratch_types=[
        pltpu.VMEM((RPW, D), jnp.float32),
        pltpu.SemaphoreType.DMA,
        pltpu.SemaphoreType.DMA,
    ],
)
def _spatial_pos_enc(row_hbm, col_hbm, out_hbm, buf, sem_r, sem_c):
    wid = lax.axis_index("s") * NC + lax.axis_index("c")
    i = wid // 2
    j0 = (wid % 2) * RPW
    base = wid * RPW

    in_r = pltpu.async_copy(
        row_hbm.at[pl.ds(i, 1)], buf.at[pl.ds(0, 1), pl.ds(0, DH)], sem_r
    )
    in_c = pltpu.async_copy(
        col_hbm.at[pl.ds(j0, RPW)], buf.at[:, pl.ds(DH, DH)], sem_c
    )

    in_r.wait()
    for k in range(DH // L):
        v = buf[0, pl.ds(k * L, L)]
        for t in range(1, RPW):
            buf[t, pl.ds(k * L, L)] = v

    in_c.wait()
    pltpu.async_copy(buf, out_hbm.at[pl.ds(base, RPW)], sem_r).wait()


def kernel(row_embed, col_embed):
    return _spatial_pos_enc(row_embed, col_embed)


# speedup vs baseline: 1.0063x; 1.0063x over previous
import functools

import jax
import jax.numpy as jnp
from jax import lax
from jax.experimental import pallas as pl
from jax.experimental.pallas import tpu as pltpu
from jax.experimental.pallas import tpu_sc as plsc

PH = 16
DH = 1024
NROWS = PH * PH
D = 2 * DH
NC = 1
NS = 16
RPW = NROWS // (NC * NS)
L = 16

_mesh = plsc.VectorSubcoreMesh(
    core_axis_name="c", subcore_axis_name="s", num_cores=NC
)


@functools.partial(
    pl.kernel,
    mesh=_mesh,
    out_type=jax.ShapeDtypeStruct((NROWS, D), jnp.float32),
    scratch_types=[
        pltpu.VMEM((RPW, D), jnp.float32),
        pltpu.SemaphoreType.DMA,
        pltpu.SemaphoreType.DMA,
    ],
)
def _spatial_pos_enc(row_hbm, col_hbm, out_hbm, buf, sem_r, sem_c):
    wid = lax.axis_index("s")
    i = wid
    j0 = 0
    base = wid * RPW

    in_r = pltpu.async_copy(
        row_hbm.at[pl.ds(i, 1)], buf.at[pl.ds(0, 1), pl.ds(0, DH)], sem_r
    )
    in_c = pltpu.async_copy(
        col_hbm.at[pl.ds(j0, RPW)], buf.at[:, pl.ds(DH, DH)], sem_c
    )

    in_r.wait()
    for k in range(DH // L):
        v = buf[0, pl.ds(k * L, L)]
        for t in range(1, RPW):
            buf[t, pl.ds(k * L, L)] = v

    in_c.wait()
    pltpu.async_copy(buf, out_hbm.at[pl.ds(base, RPW)], sem_r).wait()


def kernel(row_embed, col_embed):
    return _spatial_pos_enc(row_embed, col_embed)
